# baseline (speedup 1.0000x reference)
import functools

import jax
import jax.numpy as jnp
from jax import lax
from jax.experimental import pallas as pl
from jax.experimental.pallas import tpu as pltpu
from jax.experimental.pallas import tpu_sc as plsc

BATCH = 16384
D = 32
NW = 32
BPW = BATCH // NW
CH = 128
NCH = BPW // CH
L = 16


def _gmf_body(items_r, users_r, items_emb_r, users_emb_r, out_r,
              it_idx, us_idx, a_rows, b_rows, out_v, sem):
    wid = lax.axis_index("s") * 2 + lax.axis_index("c")
    base = wid * BPW

    idx_cps = []
    for j in range(NCH):
        off = base + j * CH
        idx_cps.append(pltpu.async_copy(
            items_r.at[pl.ds(off, CH)], it_idx.at[pl.ds(j * CH, CH)], sem))
        idx_cps.append(pltpu.async_copy(
            users_r.at[pl.ds(off, CH)], us_idx.at[pl.ds(j * CH, CH)], sem))
    for cp in idx_cps:
        cp.wait()

    row_cps = []
    for j in range(NCH):
        sl = pl.ds(j * CH, CH)
        row_cps.append(pltpu.async_copy(
            items_emb_r.at[it_idx.at[sl]], a_rows.at[sl], sem))
        row_cps.append(pltpu.async_copy(
            users_emb_r.at[us_idx.at[sl]], b_rows.at[sl], sem))
    for cp in row_cps:
        cp.wait()

    lane = lax.iota(jnp.int32, L)

    def group(g, carry):
        ridx = g * L + lane
        acc = jnp.zeros((L,), jnp.float32)
        for d in range(D):
            cd = lax.full((L,), d, jnp.int32)
            pa = plsc.load_gather(a_rows, [ridx, cd])
            pb = plsc.load_gather(b_rows, [ridx, cd])
            acc = acc + pa * pb
        sig = 1.0 / (1.0 + jnp.exp(-acc))
        out_v[pl.ds(g * L, L)] = sig
        return carry

    lax.fori_loop(0, BPW // L, group, 0)

    pltpu.sync_copy(out_v, out_r.at[pl.ds(base, BPW)])


@jax.jit
def _gmf(items, users, items_embedding, users_embedding):
    mesh = plsc.VectorSubcoreMesh(core_axis_name="c", subcore_axis_name="s")
    kfn = functools.partial(
        pl.kernel,
        mesh=mesh,
        out_type=jax.ShapeDtypeStruct((BATCH,), jnp.float32),
        scratch_types=[
            pltpu.VMEM((BPW,), jnp.int32),
            pltpu.VMEM((BPW,), jnp.int32),
            pltpu.VMEM((BPW, D), jnp.float32),
            pltpu.VMEM((BPW, D), jnp.float32),
            pltpu.VMEM((BPW,), jnp.float32),
            pltpu.SemaphoreType.DMA,
        ],
        compiler_params=pltpu.CompilerParams(
            needs_layout_passes=False, use_tc_tiling_on_sc=False),
    )(_gmf_body)
    return kfn(items, users, items_embedding, users_embedding)


def kernel(items, users, items_embedding, users_embedding):
    return _gmf(items.astype(jnp.int32), users.astype(jnp.int32),
                items_embedding, users_embedding)


# speedup vs baseline: 1.4905x; 1.4905x over previous
import functools

import jax
import jax.numpy as jnp
from jax import lax
from jax.experimental import pallas as pl
from jax.experimental.pallas import tpu as pltpu
from jax.experimental.pallas import tpu_sc as plsc

BATCH = 16384
D = 32
NW = 32
BPW = BATCH // NW
CH = 256
NCH = BPW // CH
L = 16


def _gmf_body(items_r, users_r, items_emb_r, users_emb_r, out_r,
              it_idx, us_idx, a_rows, b_rows, out_v, sem):
    wid = lax.axis_index("s") * 2 + lax.axis_index("c")
    base = wid * BPW

    cp1 = pltpu.async_copy(items_r.at[pl.ds(base, BPW)], it_idx, sem)
    cp2 = pltpu.async_copy(users_r.at[pl.ds(base, BPW)], us_idx, sem)
    cp1.wait()
    cp2.wait()

    lane = lax.iota(jnp.int32, L)

    def chunk(c, carry):
        def fire(g, carry):
            va = it_idx[pl.ds(c * CH + g * L, L)]
            vb = us_idx[pl.ds(c * CH + g * L, L)]
            for j in range(L):
                pltpu.async_copy(items_emb_r.at[pl.ds(va[j], 1)],
                                 a_rows.at[pl.ds(g * L + j, 1)], sem)
                pltpu.async_copy(users_emb_r.at[pl.ds(vb[j], 1)],
                                 b_rows.at[pl.ds(g * L + j, 1)], sem)
            return carry

        lax.fori_loop(0, CH // L, fire, 0)

        def drain(r, carry):
            pltpu.make_async_copy(items_emb_r.at[pl.ds(0, 1)],
                                  a_rows.at[pl.ds(r, 1)], sem).wait()
            pltpu.make_async_copy(users_emb_r.at[pl.ds(0, 1)],
                                  b_rows.at[pl.ds(r, 1)], sem).wait()
            return carry

        lax.fori_loop(0, CH, drain, 0)

        def group(g, carry):
            ridx = g * L + lane
            acc = jnp.zeros((L,), jnp.float32)
            for d in range(D):
                cd = lax.full((L,), d, jnp.int32)
                pa = plsc.load_gather(a_rows, [ridx, cd])
                pb = plsc.load_gather(b_rows, [ridx, cd])
                acc = acc + pa * pb
            sig = 1.0 / (1.0 + jnp.exp(-acc))
            out_v[pl.ds(c * CH + g * L, L)] = sig
            return carry

        lax.fori_loop(0, CH // L, group, 0)
        return carry

    lax.fori_loop(0, NCH, chunk, 0)

    pltpu.sync_copy(out_v, out_r.at[pl.ds(base, BPW)])


@jax.jit
def _gmf(items, users, items_embedding, users_embedding):
    mesh = plsc.VectorSubcoreMesh(core_axis_name="c", subcore_axis_name="s")
    kfn = functools.partial(
        pl.kernel,
        mesh=mesh,
        out_type=jax.ShapeDtypeStruct((BATCH,), jnp.float32),
        scratch_types=[
            pltpu.VMEM((BPW,), jnp.int32),
            pltpu.VMEM((BPW,), jnp.int32),
            pltpu.VMEM((CH, D), jnp.float32),
            pltpu.VMEM((CH, D), jnp.float32),
            pltpu.VMEM((BPW,), jnp.float32),
            pltpu.SemaphoreType.DMA,
        ],
        compiler_params=pltpu.CompilerParams(needs_layout_passes=False),
    )(_gmf_body)
    return kfn(items, users, items_embedding, users_embedding)


def kernel(items, users, items_embedding, users_embedding):
    return _gmf(items.astype(jnp.int32), users.astype(jnp.int32),
                items_embedding, users_embedding)
